# baseline (speedup 1.0000x reference)
import functools

import jax
import jax.numpy as jnp
from jax.experimental import pallas as pl
from jax.experimental.pallas import tpu as pltpu

F32 = jnp.float32

N_NODES = 50000
N_EDGES = 800000

EDGE_BLK = 3200
NODE_BLK = 2000


def _relu(h):
    return jnp.maximum(h, 0.0)


def _row(b):
    return b.reshape(1, -1)


def _enc_body(x_ref, w_ref, b_ref, o_ref):
    o_ref[...] = _relu(
        jnp.dot(x_ref[...], w_ref[...], preferred_element_type=F32) + b_ref[...])


def _enc_x(x, w, b):
    n = x.shape[0]
    grid = (n // NODE_BLK,)
    return pl.pallas_call(
        _enc_body,
        grid=grid,
        in_specs=[
            pl.BlockSpec((NODE_BLK, x.shape[1]), lambda i: (i, 0)),
            pl.BlockSpec(w.shape, lambda i: (0, 0)),
            pl.BlockSpec((1, b.shape[-1]), lambda i: (0, 0)),
        ],
        out_specs=pl.BlockSpec((NODE_BLK, w.shape[1]), lambda i: (i, 0)),
        out_shape=jax.ShapeDtypeStruct((n, w.shape[1]), F32),
    )(x, w, _row(b))


def _edge1_body(gs_ref, gd_ref, f_ref, wpe_ref, bpe_ref, ws_ref, wd_ref,
                wp_ref, b1_ref, w2_ref, b2_ref, w3_ref, b3_ref, w4_ref,
                b4_ref, e_ref):
    pe = _relu(f_ref[...] * wpe_ref[...] + bpe_ref[...])
    h = (jnp.dot(gs_ref[...], ws_ref[...], preferred_element_type=F32)
         + jnp.dot(gd_ref[...], wd_ref[...], preferred_element_type=F32)
         + jnp.dot(pe, wp_ref[...], preferred_element_type=F32)
         + b1_ref[...])
    h = _relu(h)
    h = _relu(jnp.dot(h, w2_ref[...], preferred_element_type=F32) + b2_ref[...])
    h = _relu(jnp.dot(h, w3_ref[...], preferred_element_type=F32) + b3_ref[...])
    e_ref[...] = jnp.dot(h, w4_ref[...], preferred_element_type=F32) + b4_ref[...]


def _edge_mlp1(gs, gd, feat, wpe, bpe, ws, wd, wp, b1eff, w2, b2, w3, b3, w4, b4):
    e_total = gs.shape[0]
    grid = (e_total // EDGE_BLK,)
    full = lambda a: pl.BlockSpec(a.shape, lambda i: (0,) * a.ndim)
    return pl.pallas_call(
        _edge1_body,
        grid=grid,
        in_specs=[
            pl.BlockSpec((EDGE_BLK, 16), lambda i: (i, 0)),
            pl.BlockSpec((EDGE_BLK, 16), lambda i: (i, 0)),
            pl.BlockSpec((EDGE_BLK, 1), lambda i: (i, 0)),
            full(wpe), full(bpe), full(ws), full(wd), full(wp), full(b1eff),
            full(w2), full(b2), full(w3), full(b3), full(w4), full(b4),
        ],
        out_specs=pl.BlockSpec((EDGE_BLK, 16), lambda i: (i, 0)),
        out_shape=jax.ShapeDtypeStruct((e_total, 16), F32),
    )(gs, gd, feat, wpe, bpe, ws, wd, wp, b1eff, w2, b2, w3, b3, w4, b4)


def _node1_body(x_ref, agg_ref, wa_ref, wb_ref, b1_ref, w2_ref, b2_ref,
                w3_ref, b3_ref, o_ref):
    h = (jnp.dot(x_ref[...], wa_ref[...], preferred_element_type=F32)
         + jnp.dot(agg_ref[...], wb_ref[...], preferred_element_type=F32)
         + b1_ref[...])
    h = _relu(h)
    h = _relu(jnp.dot(h, w2_ref[...], preferred_element_type=F32) + b2_ref[...])
    h = jnp.dot(h, w3_ref[...], preferred_element_type=F32) + b3_ref[...]
    o_ref[...] = jnp.concatenate([h, x_ref[...]], axis=-1)


def _node_mlp1(x, agg, wa, wb, b1eff, w2, b2, w3, b3):
    n = x.shape[0]
    grid = (n // NODE_BLK,)
    full = lambda a: pl.BlockSpec(a.shape, lambda i: (0,) * a.ndim)
    return pl.pallas_call(
        _node1_body,
        grid=grid,
        in_specs=[
            pl.BlockSpec((NODE_BLK, 16), lambda i: (i, 0)),
            pl.BlockSpec((NODE_BLK, 16), lambda i: (i, 0)),
            full(wa), full(wb), full(b1eff), full(w2), full(b2), full(w3),
            full(b3),
        ],
        out_specs=pl.BlockSpec((NODE_BLK, 32), lambda i: (i, 0)),
        out_shape=jax.ShapeDtypeStruct((n, 32), F32),
    )(x, agg, wa, wb, b1eff, w2, b2, w3, b3)


def _edge2_body(gs_ref, gd_ref, f_ref, wae_ref, bae_ref, ws_ref, wd_ref,
                wp_ref, b1_ref, w2_ref, b2_ref, w3_ref, b3_ref, w4_ref,
                b4_ref, wd1_ref, bd1_ref, wd2_ref, bd2_ref, e_ref, eo_ref):
    ae = _relu(f_ref[...] * wae_ref[...] + bae_ref[...])
    h = (jnp.dot(gs_ref[...], ws_ref[...], preferred_element_type=F32)
         + jnp.dot(gd_ref[...], wd_ref[...], preferred_element_type=F32)
         + jnp.dot(ae, wp_ref[...], preferred_element_type=F32)
         + b1_ref[...])
    h = _relu(h)
    h = _relu(jnp.dot(h, w2_ref[...], preferred_element_type=F32) + b2_ref[...])
    h = _relu(jnp.dot(h, w3_ref[...], preferred_element_type=F32) + b3_ref[...])
    e = jnp.dot(h, w4_ref[...], preferred_element_type=F32) + b4_ref[...]
    e_ref[...] = e
    d = _relu(jnp.dot(e, wd1_ref[...], preferred_element_type=F32) + bd1_ref[...])
    d = jnp.dot(d, wd2_ref[...], preferred_element_type=F32) + bd2_ref[...]
    eo_ref[...] = 1.0 / (1.0 + jnp.exp(-d))


def _edge_mlp2(gs, gd, feat, wae, bae, ws, wd, wp, b1, w2, b2, w3, b3, w4, b4,
               wd1, bd1, wd2, bd2):
    e_total = gs.shape[0]
    grid = (e_total // EDGE_BLK,)
    full = lambda a: pl.BlockSpec(a.shape, lambda i: (0,) * a.ndim)
    return pl.pallas_call(
        _edge2_body,
        grid=grid,
        in_specs=[
            pl.BlockSpec((EDGE_BLK, 32), lambda i: (i, 0)),
            pl.BlockSpec((EDGE_BLK, 32), lambda i: (i, 0)),
            pl.BlockSpec((EDGE_BLK, 1), lambda i: (i, 0)),
            full(wae), full(bae), full(ws), full(wd), full(wp), full(b1),
            full(w2), full(b2), full(w3), full(b3), full(w4), full(b4),
            full(wd1), full(bd1), full(wd2), full(bd2),
        ],
        out_specs=[
            pl.BlockSpec((EDGE_BLK, 16), lambda i: (i, 0)),
            pl.BlockSpec((EDGE_BLK, 1), lambda i: (i, 0)),
        ],
        out_shape=[
            jax.ShapeDtypeStruct((e_total, 16), F32),
            jax.ShapeDtypeStruct((e_total, 1), F32),
        ],
    )(gs, gd, feat, wae, bae, ws, wd, wp, b1, w2, b2, w3, b3, w4, b4,
      wd1, bd1, wd2, bd2)


def _node2_body(x_ref, agg_ref, wa_ref, wb_ref, b1_ref, w2_ref, b2_ref,
                w3_ref, b3_ref, wn1_ref, bn1_ref, wn2_ref, bn2_ref, o_ref):
    h = (jnp.dot(x_ref[...], wa_ref[...], preferred_element_type=F32)
         + jnp.dot(agg_ref[...], wb_ref[...], preferred_element_type=F32)
         + b1_ref[...])
    h = _relu(h)
    h = _relu(jnp.dot(h, w2_ref[...], preferred_element_type=F32) + b2_ref[...])
    h = jnp.dot(h, w3_ref[...], preferred_element_type=F32) + b3_ref[...]
    d = _relu(jnp.dot(h, wn1_ref[...], preferred_element_type=F32) + bn1_ref[...])
    o_ref[...] = jnp.dot(d, wn2_ref[...], preferred_element_type=F32) + bn2_ref[...]


def _node_mlp2(x, agg, wa, wb, b1, w2, b2, w3, b3, wn1, bn1, wn2, bn2):
    n = x.shape[0]
    grid = (n // NODE_BLK,)
    full = lambda a: pl.BlockSpec(a.shape, lambda i: (0,) * a.ndim)
    return pl.pallas_call(
        _node2_body,
        grid=grid,
        in_specs=[
            pl.BlockSpec((NODE_BLK, 32), lambda i: (i, 0)),
            pl.BlockSpec((NODE_BLK, 16), lambda i: (i, 0)),
            full(wa), full(wb), full(b1), full(w2), full(b2), full(w3),
            full(b3), full(wn1), full(bn1), full(wn2), full(bn2),
        ],
        out_specs=pl.BlockSpec((NODE_BLK, 6), lambda i: (i, 0)),
        out_shape=jax.ShapeDtypeStruct((n, 6), F32),
    )(x, agg, wa, wb, b1, w2, b2, w3, b3, wn1, bn1, wn2, bn2)


def kernel(x, node_indexes_for_prediction_edges, prediction_edges_features,
           prediction_global_features, node_indexes_for_association_edges,
           association_edges_features, params):
    p = params

    w_enc, b_enc = p["node_enc"][0]
    wpe, bpe = p["pred_edge_enc"][0]
    wg, bg = p["glob_enc"][0]
    wae, bae = p["assoc_edge_enc"][0]

    u = _relu(prediction_global_features @ wg + bg)

    (w1, b1), (w2, b2), (w3, b3), (w4, b4) = p["tgl1_edge"]
    ws1, wd1_, wp1, wu1 = w1[0:16], w1[16:32], w1[32:48], w1[48:64]
    b1eff = _row(b1) + u @ wu1

    (nw1, nb1), (nw2, nb2), (nw3, nb3) = p["tgl1_node"]
    nwa1, nwb1, nwu1 = nw1[0:16], nw1[16:32], nw1[32:48]
    nb1eff = _row(nb1) + u @ nwu1

    (v1, c1), (v2, c2), (v3, c3), (v4, c4) = p["tgl2_edge"]
    vs1, vd1, vp1 = v1[0:32], v1[32:64], v1[64:80]

    (mw1, mb1), (mw2, mb2), (mw3, mb3) = p["tgl2_node"]
    mwa1, mwb1 = mw1[0:32], mw1[32:48]

    (dn1, dbn1), (dn2, dbn2) = p["node_dec"]
    (de1, dbe1), (de2, dbe2) = p["edge_dec"]

    pe_src = node_indexes_for_prediction_edges[0]
    pe_dst = node_indexes_for_prediction_edges[1]
    ae_src = node_indexes_for_association_edges[0]
    ae_dst = node_indexes_for_association_edges[1]

    enc_x = _enc_x(x, w_enc, b_enc)

    gs1 = enc_x[pe_src]
    gd1 = enc_x[pe_dst]
    e1 = _edge_mlp1(gs1, gd1, prediction_edges_features,
                    _row(wpe), _row(bpe), ws1, wd1_, wp1, b1eff,
                    w2, _row(b2), w3, _row(b3), w4, _row(b4))

    agg1 = jax.ops.segment_sum(e1, pe_dst, num_segments=N_NODES)

    merged = _node_mlp1(enc_x, agg1, nwa1, nwb1, nb1eff,
                        nw2, _row(nb2), nw3, _row(nb3))

    gs2 = merged[ae_src]
    gd2 = merged[ae_dst]
    e2, edges_out = _edge_mlp2(gs2, gd2, association_edges_features,
                               _row(wae), _row(bae), vs1, vd1, vp1, _row(c1),
                               v2, _row(c2), v3, _row(c3), v4, _row(c4),
                               de1, _row(dbe1), de2, _row(dbe2))

    agg2 = jax.ops.segment_sum(e2, ae_dst, num_segments=N_NODES)

    nodes_out = _node_mlp2(merged, agg2, mwa1, mwb1, _row(mb1),
                           mw2, _row(mb2), mw3, _row(mb3),
                           dn1, _row(dbn1), dn2, _row(dbn2))

    return (nodes_out, edges_out)


# speedup vs baseline: 2.9303x; 2.9303x over previous
import functools

import jax
import jax.numpy as jnp
from jax import lax
from jax.experimental import pallas as pl
from jax.experimental.pallas import tpu as pltpu
from jax.experimental.pallas import tpu_sc as plsc

F32 = jnp.float32
I32 = jnp.int32

N_NODES = 50000
N_EDGES = 800000

NC = 2
NS = 16
NW = NC * NS

E_PAD = 802816
PER_W = E_PAD // NW
IDX_ROWS = PER_W // 128

E_PER_GW = N_EDGES // 16
G_MAIN = E_PER_GW // 640
G_TAIL = E_PER_GW - G_MAIN * 640

Z_PAD = 50048
DUMP_ROW = N_NODES

EDGE_BLK = 3136
NODE_BLK = 2000


def _sc_gather2(table, idx_src, idx_dst):
    d = table.shape[1]
    mesh = plsc.VectorSubcoreMesh(core_axis_name="c", subcore_axis_name="s")

    @functools.partial(
        pl.kernel,
        out_type=[jax.ShapeDtypeStruct((N_EDGES, d), F32),
                  jax.ShapeDtypeStruct((N_EDGES, d), F32)],
        mesh=mesh,
        scratch_types=[
            pltpu.VMEM((E_PER_GW,), I32),
            pltpu.VMEM((640, d), F32),
            pltpu.VMEM((G_TAIL, d), F32),
            pltpu.SemaphoreType.DMA,
        ],
        compiler_params=pltpu.CompilerParams(use_tc_tiling_on_sc=False),
    )
    def k(table_hbm, isrc_hbm, idst_hbm, osrc_hbm, odst_hbm,
          idx_v, rows_v, tail_v, sem):
        wid = lax.axis_index("s") * NC + lax.axis_index("c")

        def run(idx_hbm, out_hbm, base):
            pltpu.sync_copy(idx_hbm.at[pl.ds(base, E_PER_GW)], idx_v)

            def body(g, carry):
                off = g * 640
                cps = [
                    pltpu.async_copy(
                        table_hbm.at[idx_v.at[pl.ds(off + b * 128, 128)]],
                        rows_v.at[pl.ds(b * 128, 128)], sem)
                    for b in range(5)
                ]
                for cp in cps:
                    cp.wait()
                pltpu.sync_copy(rows_v, out_hbm.at[pl.ds(base + off, 640)])
                return carry

            lax.fori_loop(0, G_MAIN, body, 0)
            pltpu.async_copy(
                table_hbm.at[idx_v.at[pl.ds(G_MAIN * 640, G_TAIL)]],
                tail_v, sem).wait()
            pltpu.sync_copy(tail_v,
                            out_hbm.at[pl.ds(base + G_MAIN * 640, G_TAIL)])

        @pl.when(wid < 16)
        def _():
            run(isrc_hbm, osrc_hbm, wid * E_PER_GW)

        @pl.when(wid >= 16)
        def _():
            run(idst_hbm, odst_hbm, (wid - 16) * E_PER_GW)

    return k(table, idx_src, idx_dst)


def _sc_scatter_add(e, idx3, zeros):
    mesh = plsc.VectorSubcoreMesh(core_axis_name="c", subcore_axis_name="s")

    @functools.partial(
        pl.kernel,
        out_type=jax.ShapeDtypeStruct((2, Z_PAD, 16), F32),
        mesh=mesh,
        scratch_types=[
            pltpu.VMEM_SHARED((Z_PAD, 16), F32),
            pltpu.VMEM((IDX_ROWS, 128), I32),
            pltpu.VMEM((512, 16), F32),
            pltpu.SemaphoreType.DMA,
        ],
        compiler_params=pltpu.CompilerParams(use_tc_tiling_on_sc=False),
    )
    def k(e_hbm, idx_hbm, z_hbm, out_hbm, shared, idx_v, rows_v, sem):
        c = lax.axis_index("c")
        s = lax.axis_index("s")
        wid = s * NC + c

        @pl.when(s == 0)
        def _():
            pltpu.sync_copy(z_hbm, shared)

        plsc.subcore_barrier()
        pltpu.sync_copy(idx_hbm.at[wid], idx_v)
        base = wid * PER_W

        def body(t, carry):
            pltpu.sync_copy(e_hbm.at[pl.ds(base + t * 512, 512)], rows_v)
            for b in range(4):
                pltpu.sync_copy(rows_v.at[pl.ds(b * 128, 128)],
                                shared.at[idx_v.at[t * 4 + b]], add=True)
            return carry

        lax.fori_loop(0, IDX_ROWS // 4, body, 0)
        plsc.subcore_barrier()
        pltpu.sync_copy(shared.at[pl.ds(s * (Z_PAD // NS), Z_PAD // NS)],
                        out_hbm.at[c, pl.ds(s * (Z_PAD // NS), Z_PAD // NS)])

    return k(e, idx3, zeros)


def _relu(h):
    return jnp.maximum(h, 0.0)


def _row(b):
    return b.reshape(1, -1)


def _enc_body(x_ref, w_ref, b_ref, o_ref):
    o_ref[...] = _relu(
        jnp.dot(x_ref[...], w_ref[...], preferred_element_type=F32) + b_ref[...])


def _enc_x(x, w, b):
    n = x.shape[0]
    grid = (n // NODE_BLK,)
    return pl.pallas_call(
        _enc_body,
        grid=grid,
        in_specs=[
            pl.BlockSpec((NODE_BLK, x.shape[1]), lambda i: (i, 0)),
            pl.BlockSpec(w.shape, lambda i: (0, 0)),
            pl.BlockSpec((1, b.shape[-1]), lambda i: (0, 0)),
        ],
        out_specs=pl.BlockSpec((NODE_BLK, w.shape[1]), lambda i: (i, 0)),
        out_shape=jax.ShapeDtypeStruct((n, w.shape[1]), F32),
    )(x, w, _row(b))


def _edge1_body(gs_ref, gd_ref, f_ref, wpe_ref, bpe_ref, ws_ref, wd_ref,
                wp_ref, b1_ref, w2_ref, b2_ref, w3_ref, b3_ref, w4_ref,
                b4_ref, e_ref):
    pe = _relu(f_ref[...] * wpe_ref[...] + bpe_ref[...])
    h = (jnp.dot(gs_ref[...], ws_ref[...], preferred_element_type=F32)
         + jnp.dot(gd_ref[...], wd_ref[...], preferred_element_type=F32)
         + jnp.dot(pe, wp_ref[...], preferred_element_type=F32)
         + b1_ref[...])
    h = _relu(h)
    h = _relu(jnp.dot(h, w2_ref[...], preferred_element_type=F32) + b2_ref[...])
    h = _relu(jnp.dot(h, w3_ref[...], preferred_element_type=F32) + b3_ref[...])
    e_ref[...] = jnp.dot(h, w4_ref[...], preferred_element_type=F32) + b4_ref[...]


def _edge_mlp1(gs, gd, feat, wpe, bpe, ws, wd, wp, b1eff, w2, b2, w3, b3, w4, b4):
    grid = (E_PAD // EDGE_BLK,)
    full = lambda a: pl.BlockSpec(a.shape, lambda i: (0,) * a.ndim)
    return pl.pallas_call(
        _edge1_body,
        grid=grid,
        in_specs=[
            pl.BlockSpec((EDGE_BLK, 16), lambda i: (i, 0)),
            pl.BlockSpec((EDGE_BLK, 16), lambda i: (i, 0)),
            pl.BlockSpec((EDGE_BLK, 1), lambda i: (i, 0)),
            full(wpe), full(bpe), full(ws), full(wd), full(wp), full(b1eff),
            full(w2), full(b2), full(w3), full(b3), full(w4), full(b4),
        ],
        out_specs=pl.BlockSpec((EDGE_BLK, 16), lambda i: (i, 0)),
        out_shape=jax.ShapeDtypeStruct((E_PAD, 16), F32),
    )(gs, gd, feat, wpe, bpe, ws, wd, wp, b1eff, w2, b2, w3, b3, w4, b4)


def _node1_body(x_ref, p0_ref, p1_ref, wa_ref, wb_ref, b1_ref, w2_ref, b2_ref,
                w3_ref, b3_ref, o_ref):
    agg = p0_ref[0] + p1_ref[0]
    h = (jnp.dot(x_ref[...], wa_ref[...], preferred_element_type=F32)
         + jnp.dot(agg, wb_ref[...], preferred_element_type=F32)
         + b1_ref[...])
    h = _relu(h)
    h = _relu(jnp.dot(h, w2_ref[...], preferred_element_type=F32) + b2_ref[...])
    h = jnp.dot(h, w3_ref[...], preferred_element_type=F32) + b3_ref[...]
    o_ref[...] = jnp.concatenate([h, x_ref[...]], axis=-1)


def _node_mlp1(x, partials, wa, wb, b1eff, w2, b2, w3, b3):
    n = x.shape[0]
    grid = (n // NODE_BLK,)
    full = lambda a: pl.BlockSpec(a.shape, lambda i: (0,) * a.ndim)
    return pl.pallas_call(
        _node1_body,
        grid=grid,
        in_specs=[
            pl.BlockSpec((NODE_BLK, 16), lambda i: (i, 0)),
            pl.BlockSpec((1, NODE_BLK, 16), lambda i: (0, i, 0)),
            pl.BlockSpec((1, NODE_BLK, 16), lambda i: (1, i, 0)),
            full(wa), full(wb), full(b1eff), full(w2), full(b2), full(w3),
            full(b3),
        ],
        out_specs=pl.BlockSpec((NODE_BLK, 32), lambda i: (i, 0)),
        out_shape=jax.ShapeDtypeStruct((n, 32), F32),
    )(x, partials, partials, wa, wb, b1eff, w2, b2, w3, b3)


def _edge2_body(gs_ref, gd_ref, f_ref, wae_ref, bae_ref, ws_ref, wd_ref,
                wp_ref, b1_ref, w2_ref, b2_ref, w3_ref, b3_ref, w4_ref,
                b4_ref, wd1_ref, bd1_ref, wd2_ref, bd2_ref, e_ref, eo_ref):
    ae = _relu(f_ref[...] * wae_ref[...] + bae_ref[...])
    h = (jnp.dot(gs_ref[...], ws_ref[...], preferred_element_type=F32)
         + jnp.dot(gd_ref[...], wd_ref[...], preferred_element_type=F32)
         + jnp.dot(ae, wp_ref[...], preferred_element_type=F32)
         + b1_ref[...])
    h = _relu(h)
    h = _relu(jnp.dot(h, w2_ref[...], preferred_element_type=F32) + b2_ref[...])
    h = _relu(jnp.dot(h, w3_ref[...], preferred_element_type=F32) + b3_ref[...])
    e = jnp.dot(h, w4_ref[...], preferred_element_type=F32) + b4_ref[...]
    e_ref[...] = e
    d = _relu(jnp.dot(e, wd1_ref[...], preferred_element_type=F32) + bd1_ref[...])
    d = jnp.dot(d, wd2_ref[...], preferred_element_type=F32) + bd2_ref[...]
    eo_ref[...] = 1.0 / (1.0 + jnp.exp(-d))


def _edge_mlp2(gs, gd, feat, wae, bae, ws, wd, wp, b1, w2, b2, w3, b3, w4, b4,
               wd1, bd1, wd2, bd2):
    grid = (E_PAD // EDGE_BLK,)
    full = lambda a: pl.BlockSpec(a.shape, lambda i: (0,) * a.ndim)
    return pl.pallas_call(
        _edge2_body,
        grid=grid,
        in_specs=[
            pl.BlockSpec((EDGE_BLK, 32), lambda i: (i, 0)),
            pl.BlockSpec((EDGE_BLK, 32), lambda i: (i, 0)),
            pl.BlockSpec((EDGE_BLK, 1), lambda i: (i, 0)),
            full(wae), full(bae), full(ws), full(wd), full(wp), full(b1),
            full(w2), full(b2), full(w3), full(b3), full(w4), full(b4),
            full(wd1), full(bd1), full(wd2), full(bd2),
        ],
        out_specs=[
            pl.BlockSpec((EDGE_BLK, 16), lambda i: (i, 0)),
            pl.BlockSpec((EDGE_BLK, 1), lambda i: (i, 0)),
        ],
        out_shape=[
            jax.ShapeDtypeStruct((E_PAD, 16), F32),
            jax.ShapeDtypeStruct((N_EDGES, 1), F32),
        ],
    )(gs, gd, feat, wae, bae, ws, wd, wp, b1, w2, b2, w3, b3, w4, b4,
      wd1, bd1, wd2, bd2)


def _node2_body(x_ref, p0_ref, p1_ref, wa_ref, wb_ref, b1_ref, w2_ref, b2_ref,
                w3_ref, b3_ref, wn1_ref, bn1_ref, wn2_ref, bn2_ref, o_ref):
    agg = p0_ref[0] + p1_ref[0]
    h = (jnp.dot(x_ref[...], wa_ref[...], preferred_element_type=F32)
         + jnp.dot(agg, wb_ref[...], preferred_element_type=F32)
         + b1_ref[...])
    h = _relu(h)
    h = _relu(jnp.dot(h, w2_ref[...], preferred_element_type=F32) + b2_ref[...])
    h = jnp.dot(h, w3_ref[...], preferred_element_type=F32) + b3_ref[...]
    d = _relu(jnp.dot(h, wn1_ref[...], preferred_element_type=F32) + bn1_ref[...])
    o_ref[...] = jnp.dot(d, wn2_ref[...], preferred_element_type=F32) + bn2_ref[...]


def _node_mlp2(x, partials, wa, wb, b1, w2, b2, w3, b3, wn1, bn1, wn2, bn2):
    n = x.shape[0]
    grid = (n // NODE_BLK,)
    full = lambda a: pl.BlockSpec(a.shape, lambda i: (0,) * a.ndim)
    return pl.pallas_call(
        _node2_body,
        grid=grid,
        in_specs=[
            pl.BlockSpec((NODE_BLK, 32), lambda i: (i, 0)),
            pl.BlockSpec((1, NODE_BLK, 16), lambda i: (0, i, 0)),
            pl.BlockSpec((1, NODE_BLK, 16), lambda i: (1, i, 0)),
            full(wa), full(wb), full(b1), full(w2), full(b2), full(w3),
            full(b3), full(wn1), full(bn1), full(wn2), full(bn2),
        ],
        out_specs=pl.BlockSpec((NODE_BLK, 6), lambda i: (i, 0)),
        out_shape=jax.ShapeDtypeStruct((n, 6), F32),
    )(x, partials, partials, wa, wb, b1, w2, b2, w3, b3, wn1, bn1, wn2, bn2)


def kernel(x, node_indexes_for_prediction_edges, prediction_edges_features,
           prediction_global_features, node_indexes_for_association_edges,
           association_edges_features, params):
    p = params

    w_enc, b_enc = p["node_enc"][0]
    wpe, bpe = p["pred_edge_enc"][0]
    wg, bg = p["glob_enc"][0]
    wae, bae = p["assoc_edge_enc"][0]

    u = _relu(prediction_global_features @ wg + bg)

    (w1, b1), (w2, b2), (w3, b3), (w4, b4) = p["tgl1_edge"]
    ws1, wd1_, wp1, wu1 = w1[0:16], w1[16:32], w1[32:48], w1[48:64]
    b1eff = _row(b1) + u @ wu1

    (nw1, nb1), (nw2, nb2), (nw3, nb3) = p["tgl1_node"]
    nwa1, nwb1, nwu1 = nw1[0:16], nw1[16:32], nw1[32:48]
    nb1eff = _row(nb1) + u @ nwu1

    (v1, c1), (v2, c2), (v3, c3), (v4, c4) = p["tgl2_edge"]
    vs1, vd1, vp1 = v1[0:32], v1[32:64], v1[64:80]

    (mw1, mb1), (mw2, mb2), (mw3, mb3) = p["tgl2_node"]
    mwa1, mwb1 = mw1[0:32], mw1[32:48]

    (dn1, dbn1), (dn2, dbn2) = p["node_dec"]
    (de1, dbe1), (de2, dbe2) = p["edge_dec"]

    pe_src = node_indexes_for_prediction_edges[0]
    pe_dst = node_indexes_for_prediction_edges[1]
    ae_src = node_indexes_for_association_edges[0]
    ae_dst = node_indexes_for_association_edges[1]

    pad = jnp.full((E_PAD - N_EDGES,), DUMP_ROW, dtype=pe_dst.dtype)
    pe_dst3 = jnp.concatenate([pe_dst, pad]).reshape(NW, IDX_ROWS, 128)
    ae_dst3 = jnp.concatenate([ae_dst, pad]).reshape(NW, IDX_ROWS, 128)
    zeros = jnp.zeros((Z_PAD, 16), F32)

    enc_x = _enc_x(x, w_enc, b_enc)

    gs1, gd1 = _sc_gather2(enc_x, pe_src, pe_dst)
    e1 = _edge_mlp1(gs1, gd1, prediction_edges_features,
                    _row(wpe), _row(bpe), ws1, wd1_, wp1, b1eff,
                    w2, _row(b2), w3, _row(b3), w4, _row(b4))

    agg1 = _sc_scatter_add(e1, pe_dst3, zeros)

    merged = _node_mlp1(enc_x, agg1, nwa1, nwb1, nb1eff,
                        nw2, _row(nb2), nw3, _row(nb3))

    gs2, gd2 = _sc_gather2(merged, ae_src, ae_dst)
    e2, edges_out = _edge_mlp2(gs2, gd2, association_edges_features,
                               _row(wae), _row(bae), vs1, vd1, vp1, _row(c1),
                               v2, _row(c2), v3, _row(c3), v4, _row(c4),
                               de1, _row(dbe1), de2, _row(dbe2))

    agg2 = _sc_scatter_add(e2, ae_dst3, zeros)

    nodes_out = _node_mlp2(merged, agg2, mwa1, mwb1, _row(mb1),
                           mw2, _row(mb2), mw3, _row(mb3),
                           dn1, _row(dbn1), dn2, _row(dbn2))

    return (nodes_out, edges_out)


# speedup vs baseline: 8.4817x; 2.8945x over previous
import functools

import jax
import jax.numpy as jnp
from jax import lax
from jax.experimental import pallas as pl
from jax.experimental.pallas import tpu as pltpu
from jax.experimental.pallas import tpu_sc as plsc

F32 = jnp.float32
I32 = jnp.int32

N_NODES = 50000
N_EDGES = 800000

NC = 2
NS = 16
NW = NC * NS

E_PAD = 802816
PER_W = E_PAD // NW
IDX_ROWS = PER_W // 128

E_PER_GW = N_EDGES // 16
G_MAIN = E_PER_GW // 640
G_TAIL = E_PER_GW - G_MAIN * 640

Z_PAD = 50048
DUMP_ROW = N_NODES

EDGE_BLK = 1568
NODE_BLK = 2000


def _sc_gather2(table, idx_src, idx_dst):
    d = table.shape[1]
    mesh = plsc.VectorSubcoreMesh(core_axis_name="c", subcore_axis_name="s")

    @functools.partial(
        pl.kernel,
        out_type=[jax.ShapeDtypeStruct((N_EDGES, d), F32),
                  jax.ShapeDtypeStruct((N_EDGES, d), F32)],
        mesh=mesh,
        scratch_types=[
            pltpu.VMEM((E_PER_GW,), I32),
            pltpu.VMEM((640, d), F32),
            pltpu.VMEM((G_TAIL, d), F32),
            pltpu.SemaphoreType.DMA,
        ],
        compiler_params=pltpu.CompilerParams(use_tc_tiling_on_sc=False),
    )
    def k(table_hbm, isrc_hbm, idst_hbm, osrc_hbm, odst_hbm,
          idx_v, rows_v, tail_v, sem):
        wid = lax.axis_index("s") * NC + lax.axis_index("c")

        def run(idx_hbm, out_hbm, base):
            pltpu.sync_copy(idx_hbm.at[pl.ds(base, E_PER_GW)], idx_v)

            def body(g, carry):
                off = g * 640
                cps = [
                    pltpu.async_copy(
                        table_hbm.at[idx_v.at[pl.ds(off + b * 128, 128)]],
                        rows_v.at[pl.ds(b * 128, 128)], sem)
                    for b in range(5)
                ]
                for cp in cps:
                    cp.wait()
                pltpu.sync_copy(rows_v, out_hbm.at[pl.ds(base + off, 640)])
                return carry

            lax.fori_loop(0, G_MAIN, body, 0)
            pltpu.async_copy(
                table_hbm.at[idx_v.at[pl.ds(G_MAIN * 640, G_TAIL)]],
                tail_v, sem).wait()
            pltpu.sync_copy(tail_v,
                            out_hbm.at[pl.ds(base + G_MAIN * 640, G_TAIL)])

        @pl.when(wid < 16)
        def _():
            run(isrc_hbm, osrc_hbm, wid * E_PER_GW)

        @pl.when(wid >= 16)
        def _():
            run(idst_hbm, odst_hbm, (wid - 16) * E_PER_GW)

    return k(table, idx_src, idx_dst)


def _sc_scatter_add(e, idx3, zeros):
    mesh = plsc.VectorSubcoreMesh(core_axis_name="c", subcore_axis_name="s")

    @functools.partial(
        pl.kernel,
        out_type=jax.ShapeDtypeStruct((2, Z_PAD, 16), F32),
        mesh=mesh,
        scratch_types=[
            pltpu.VMEM_SHARED((Z_PAD, 16), F32),
            pltpu.VMEM((IDX_ROWS, 128), I32),
            pltpu.VMEM((512, 16), F32),
            pltpu.SemaphoreType.DMA,
        ],
        compiler_params=pltpu.CompilerParams(use_tc_tiling_on_sc=False),
    )
    def k(e_hbm, idx_hbm, z_hbm, out_hbm, shared, idx_v, rows_v, sem):
        c = lax.axis_index("c")
        s = lax.axis_index("s")
        wid = s * NC + c

        @pl.when(s == 0)
        def _():
            pltpu.sync_copy(z_hbm, shared)

        plsc.subcore_barrier()
        pltpu.sync_copy(idx_hbm.at[wid], idx_v)
        base = wid * PER_W

        def body(t, carry):
            pltpu.sync_copy(e_hbm.at[pl.ds(base + t * 512, 512)], rows_v)
            for b in range(4):
                pltpu.sync_copy(rows_v.at[pl.ds(b * 128, 128)],
                                shared.at[idx_v.at[t * 4 + b]], add=True)
            return carry

        lax.fori_loop(0, IDX_ROWS // 4, body, 0)
        plsc.subcore_barrier()
        pltpu.sync_copy(shared.at[pl.ds(s * (Z_PAD // NS), Z_PAD // NS)],
                        out_hbm.at[c, pl.ds(s * (Z_PAD // NS), Z_PAD // NS)])

    return k(e, idx3, zeros)


def _relu(h):
    return jnp.maximum(h, 0.0)


def _row(b):
    return b.reshape(1, -1)


def _enc_body(x_ref, w_ref, b_ref, o_ref):
    o_ref[...] = _relu(
        jnp.dot(x_ref[...], w_ref[...], preferred_element_type=F32) + b_ref[...])


def _enc_x(x, w, b):
    n = x.shape[0]
    grid = (n // NODE_BLK,)
    return pl.pallas_call(
        _enc_body,
        grid=grid,
        in_specs=[
            pl.BlockSpec((NODE_BLK, x.shape[1]), lambda i: (i, 0)),
            pl.BlockSpec(w.shape, lambda i: (0, 0)),
            pl.BlockSpec((1, b.shape[-1]), lambda i: (0, 0)),
        ],
        out_specs=pl.BlockSpec((NODE_BLK, w.shape[1]), lambda i: (i, 0)),
        out_shape=jax.ShapeDtypeStruct((n, w.shape[1]), F32),
    )(x, w, _row(b))


def _edge1_body(gs_ref, gd_ref, f_ref, r0_ref, wpe_ref, bpe_ref, ws_ref,
                wd_ref, wp_ref, b1_ref, w2_ref, b2_ref, w3_ref, b3_ref,
                w4_ref, b4_ref, e_ref):
    f_rep = jnp.dot(f_ref[...], r0_ref[...], preferred_element_type=F32,
                    precision=jax.lax.Precision.HIGHEST)
    pe = _relu(f_rep * wpe_ref[...] + bpe_ref[...])
    h = (jnp.dot(gs_ref[...], ws_ref[...], preferred_element_type=F32)
         + jnp.dot(gd_ref[...], wd_ref[...], preferred_element_type=F32)
         + jnp.dot(pe, wp_ref[...], preferred_element_type=F32)
         + b1_ref[...])
    h = _relu(h)
    h = _relu(jnp.dot(h, w2_ref[...], preferred_element_type=F32) + b2_ref[...])
    h = _relu(jnp.dot(h, w3_ref[...], preferred_element_type=F32) + b3_ref[...])
    e_ref[...] = jnp.dot(h, w4_ref[...], preferred_element_type=F32) + b4_ref[...]


def _edge_mlp1(gs, gd, feat, r0, wpe, bpe, ws, wd, wp, b1eff, w2, b2, w3, b3,
               w4, b4):
    grid = (E_PAD // 8 // EDGE_BLK,)
    full = lambda a: pl.BlockSpec(a.shape, lambda i: (0,) * a.ndim)
    return pl.pallas_call(
        _edge1_body,
        grid=grid,
        in_specs=[
            pl.BlockSpec((EDGE_BLK, 128), lambda i: (i, 0)),
            pl.BlockSpec((EDGE_BLK, 128), lambda i: (i, 0)),
            pl.BlockSpec((EDGE_BLK, 8), lambda i: (i, 0)),
            full(r0), full(wpe), full(bpe), full(ws), full(wd), full(wp),
            full(b1eff), full(w2), full(b2), full(w3), full(b3), full(w4),
            full(b4),
        ],
        out_specs=pl.BlockSpec((EDGE_BLK, 128), lambda i: (i, 0)),
        out_shape=jax.ShapeDtypeStruct((E_PAD // 8, 128), F32),
    )(gs, gd, feat, r0, wpe, bpe, ws, wd, wp, b1eff, w2, b2, w3, b3, w4, b4)


def _node1_body(x_ref, p0_ref, p1_ref, wa_ref, wb_ref, b1_ref, w2_ref, b2_ref,
                w3_ref, b3_ref, o_ref):
    agg = p0_ref[0] + p1_ref[0]
    h = (jnp.dot(x_ref[...], wa_ref[...], preferred_element_type=F32)
         + jnp.dot(agg, wb_ref[...], preferred_element_type=F32)
         + b1_ref[...])
    h = _relu(h)
    h = _relu(jnp.dot(h, w2_ref[...], preferred_element_type=F32) + b2_ref[...])
    h = jnp.dot(h, w3_ref[...], preferred_element_type=F32) + b3_ref[...]
    o_ref[...] = jnp.concatenate([h, x_ref[...]], axis=-1)


def _node_mlp1(x, partials, wa, wb, b1eff, w2, b2, w3, b3):
    n = x.shape[0]
    grid = (n // NODE_BLK,)
    full = lambda a: pl.BlockSpec(a.shape, lambda i: (0,) * a.ndim)
    return pl.pallas_call(
        _node1_body,
        grid=grid,
        in_specs=[
            pl.BlockSpec((NODE_BLK, 16), lambda i: (i, 0)),
            pl.BlockSpec((1, NODE_BLK, 16), lambda i: (0, i, 0)),
            pl.BlockSpec((1, NODE_BLK, 16), lambda i: (1, i, 0)),
            full(wa), full(wb), full(b1eff), full(w2), full(b2), full(w3),
            full(b3),
        ],
        out_specs=pl.BlockSpec((NODE_BLK, 32), lambda i: (i, 0)),
        out_shape=jax.ShapeDtypeStruct((n, 32), F32),
    )(x, partials, partials, wa, wb, b1eff, w2, b2, w3, b3)


def _edge2_body(gs_ref, gd_ref, f_ref, r0_ref, wae_ref, bae_ref, ws_ref,
                wd_ref, wp_ref, b1_ref, w2_ref, b2_ref, w3_ref, b3_ref,
                w4_ref, b4_ref, wd1_ref, bd1_ref, wd2_ref, bd2_ref,
                e_ref, eo_ref):
    f_rep = jnp.dot(f_ref[...], r0_ref[...], preferred_element_type=F32,
                    precision=jax.lax.Precision.HIGHEST)
    ae = _relu(f_rep * wae_ref[...] + bae_ref[...])
    h = (jnp.dot(gs_ref[...], ws_ref[...], preferred_element_type=F32)
         + jnp.dot(gd_ref[...], wd_ref[...], preferred_element_type=F32)
         + jnp.dot(ae, wp_ref[...], preferred_element_type=F32)
         + b1_ref[...])
    h = _relu(h)
    h = _relu(jnp.dot(h, w2_ref[...], preferred_element_type=F32) + b2_ref[...])
    h = _relu(jnp.dot(h, w3_ref[...], preferred_element_type=F32) + b3_ref[...])
    e = jnp.dot(h, w4_ref[...], preferred_element_type=F32) + b4_ref[...]
    e_ref[...] = e
    d = _relu(jnp.dot(e, wd1_ref[...], preferred_element_type=F32) + bd1_ref[...])
    d = jnp.dot(d, wd2_ref[...], preferred_element_type=F32) + bd2_ref[...]
    eo_ref[...] = 1.0 / (1.0 + jnp.exp(-d))


def _edge_mlp2(gs, gd, feat, r0, wae, bae, ws, wd, wp, b1, w2, b2, w3, b3,
               w4, b4, wd1, bd1, wd2, bd2):
    grid = (E_PAD // 8 // EDGE_BLK,)
    full = lambda a: pl.BlockSpec(a.shape, lambda i: (0,) * a.ndim)
    return pl.pallas_call(
        _edge2_body,
        grid=grid,
        in_specs=[
            pl.BlockSpec((EDGE_BLK, 256), lambda i: (i, 0)),
            pl.BlockSpec((EDGE_BLK, 256), lambda i: (i, 0)),
            pl.BlockSpec((EDGE_BLK, 8), lambda i: (i, 0)),
            full(r0), full(wae), full(bae), full(ws), full(wd), full(wp),
            full(b1), full(w2), full(b2), full(w3), full(b3), full(w4),
            full(b4), full(wd1), full(bd1), full(wd2), full(bd2),
        ],
        out_specs=[
            pl.BlockSpec((EDGE_BLK, 128), lambda i: (i, 0)),
            pl.BlockSpec((EDGE_BLK, 8), lambda i: (i, 0)),
        ],
        out_shape=[
            jax.ShapeDtypeStruct((E_PAD // 8, 128), F32),
            jax.ShapeDtypeStruct((E_PAD // 8, 8), F32),
        ],
    )(gs, gd, feat, r0, wae, bae, ws, wd, wp, b1, w2, b2, w3, b3, w4, b4,
      wd1, bd1, wd2, bd2)


def _node2_body(x_ref, p0_ref, p1_ref, wa_ref, wb_ref, b1_ref, w2_ref, b2_ref,
                w3_ref, b3_ref, wn1_ref, bn1_ref, wn2_ref, bn2_ref, o_ref):
    agg = p0_ref[0] + p1_ref[0]
    h = (jnp.dot(x_ref[...], wa_ref[...], preferred_element_type=F32)
         + jnp.dot(agg, wb_ref[...], preferred_element_type=F32)
         + b1_ref[...])
    h = _relu(h)
    h = _relu(jnp.dot(h, w2_ref[...], preferred_element_type=F32) + b2_ref[...])
    h = jnp.dot(h, w3_ref[...], preferred_element_type=F32) + b3_ref[...]
    d = _relu(jnp.dot(h, wn1_ref[...], preferred_element_type=F32) + bn1_ref[...])
    o_ref[...] = jnp.dot(d, wn2_ref[...], preferred_element_type=F32) + bn2_ref[...]


def _node_mlp2(x, partials, wa, wb, b1, w2, b2, w3, b3, wn1, bn1, wn2, bn2):
    n = x.shape[0]
    grid = (n // NODE_BLK,)
    full = lambda a: pl.BlockSpec(a.shape, lambda i: (0,) * a.ndim)
    return pl.pallas_call(
        _node2_body,
        grid=grid,
        in_specs=[
            pl.BlockSpec((NODE_BLK, 32), lambda i: (i, 0)),
            pl.BlockSpec((1, NODE_BLK, 16), lambda i: (0, i, 0)),
            pl.BlockSpec((1, NODE_BLK, 16), lambda i: (1, i, 0)),
            full(wa), full(wb), full(b1), full(w2), full(b2), full(w3),
            full(b3), full(wn1), full(bn1), full(wn2), full(bn2),
        ],
        out_specs=pl.BlockSpec((NODE_BLK, 6), lambda i: (i, 0)),
        out_shape=jax.ShapeDtypeStruct((n, 6), F32),
    )(x, partials, partials, wa, wb, b1, w2, b2, w3, b3, wn1, bn1, wn2, bn2)


def kernel(x, node_indexes_for_prediction_edges, prediction_edges_features,
           prediction_global_features, node_indexes_for_association_edges,
           association_edges_features, params):
    p = params

    w_enc, b_enc = p["node_enc"][0]
    wpe, bpe = p["pred_edge_enc"][0]
    wg, bg = p["glob_enc"][0]
    wae, bae = p["assoc_edge_enc"][0]

    u = _relu(prediction_global_features @ wg + bg)

    (w1, b1), (w2, b2), (w3, b3), (w4, b4) = p["tgl1_edge"]
    ws1, wd1_, wp1, wu1 = w1[0:16], w1[16:32], w1[32:48], w1[48:64]
    b1eff = _row(b1) + u @ wu1

    (nw1, nb1), (nw2, nb2), (nw3, nb3) = p["tgl1_node"]
    nwa1, nwb1, nwu1 = nw1[0:16], nw1[16:32], nw1[32:48]
    nb1eff = _row(nb1) + u @ nwu1

    (v1, c1), (v2, c2), (v3, c3), (v4, c4) = p["tgl2_edge"]
    vs1, vd1, vp1 = v1[0:32], v1[32:64], v1[64:80]

    (mw1, mb1), (mw2, mb2), (mw3, mb3) = p["tgl2_node"]
    mwa1, mwb1 = mw1[0:32], mw1[32:48]

    (dn1, dbn1), (dn2, dbn2) = p["node_dec"]
    (de1, dbe1), (de2, dbe2) = p["edge_dec"]

    pe_src = node_indexes_for_prediction_edges[0]
    pe_dst = node_indexes_for_prediction_edges[1]
    ae_src = node_indexes_for_association_edges[0]
    ae_dst = node_indexes_for_association_edges[1]

    pad = jnp.full((E_PAD - N_EDGES,), DUMP_ROW, dtype=pe_dst.dtype)
    pe_dst3 = jnp.concatenate([pe_dst, pad]).reshape(NW, IDX_ROWS, 128)
    ae_dst3 = jnp.concatenate([ae_dst, pad]).reshape(NW, IDX_ROWS, 128)
    zeros = jnp.zeros((Z_PAD, 16), F32)

    enc_x = _enc_x(x, w_enc, b_enc)

    k8 = lambda w: jnp.kron(jnp.eye(8, dtype=F32), w)
    t8 = lambda b: jnp.tile(b.reshape(1, -1), (1, 8))
    r0 = jnp.kron(jnp.eye(8, dtype=F32), jnp.ones((1, 16), F32))

    gs1, gd1 = _sc_gather2(enc_x, pe_src, pe_dst)
    e1 = _edge_mlp1(gs1.reshape(N_EDGES // 8, 128),
                    gd1.reshape(N_EDGES // 8, 128),
                    prediction_edges_features.reshape(N_EDGES // 8, 8),
                    r0, t8(wpe), t8(bpe), k8(ws1), k8(wd1_), k8(wp1),
                    t8(b1eff), k8(w2), t8(b2), k8(w3), t8(b3), k8(w4),
                    t8(b4))

    agg1 = _sc_scatter_add(e1.reshape(E_PAD, 16), pe_dst3, zeros)

    merged = _node_mlp1(enc_x, agg1, nwa1, nwb1, nb1eff,
                        nw2, _row(nb2), nw3, _row(nb3))

    gs2, gd2 = _sc_gather2(merged, ae_src, ae_dst)
    e2, eo = _edge_mlp2(gs2.reshape(N_EDGES // 8, 256),
                        gd2.reshape(N_EDGES // 8, 256),
                        association_edges_features.reshape(N_EDGES // 8, 8),
                        r0, t8(wae), t8(bae), k8(vs1), k8(vd1), k8(vp1),
                        t8(c1), k8(v2), t8(c2), k8(v3), t8(c3), k8(v4),
                        t8(c4), k8(de1), t8(dbe1), k8(de2), t8(dbe2))
    edges_out = eo.reshape(E_PAD, 1)[:N_EDGES]

    agg2 = _sc_scatter_add(e2.reshape(E_PAD, 16), ae_dst3, zeros)

    nodes_out = _node_mlp2(merged, agg2, mwa1, mwb1, _row(mb1),
                           mw2, _row(mb2), mw3, _row(mb3),
                           dn1, _row(dbn1), dn2, _row(dbn2))

    return (nodes_out, edges_out)


# speedup vs baseline: 9.4369x; 1.1126x over previous
import functools

import jax
import jax.numpy as jnp
from jax import lax
from jax.experimental import pallas as pl
from jax.experimental.pallas import tpu as pltpu
from jax.experimental.pallas import tpu_sc as plsc

F32 = jnp.float32
I32 = jnp.int32

N_NODES = 50000
N_EDGES = 800000

NC = 2
NS = 16
NW = NC * NS

E_PAD = 802816
PER_W = E_PAD // NW
IDX_ROWS = PER_W // 128

E_PER_GW = N_EDGES // 16
G_MAIN = E_PER_GW // 640
G_TAIL = E_PER_GW - G_MAIN * 640

Z_PAD = 50048
DUMP_ROW = N_NODES

EDGE_BLK = 1568
NODE_BLK = 2000


def _sc_gather2(table, idx_src, idx_dst):
    d = table.shape[1]
    mesh = plsc.VectorSubcoreMesh(core_axis_name="c", subcore_axis_name="s")

    @functools.partial(
        pl.kernel,
        out_type=[jax.ShapeDtypeStruct((N_EDGES, d), F32),
                  jax.ShapeDtypeStruct((N_EDGES, d), F32)],
        mesh=mesh,
        scratch_types=[
            pltpu.VMEM((E_PER_GW,), I32),
            pltpu.VMEM((640, d), F32),
            pltpu.VMEM((640, d), F32),
            pltpu.VMEM((G_TAIL, d), F32),
            pltpu.SemaphoreType.DMA,
            pltpu.SemaphoreType.DMA,
        ],
        compiler_params=pltpu.CompilerParams(use_tc_tiling_on_sc=False),
    )
    def k(table_hbm, isrc_hbm, idst_hbm, osrc_hbm, odst_hbm,
          idx_v, rows0_v, rows1_v, tail_v, sem0, sem1):
        wid = lax.axis_index("s") * NC + lax.axis_index("c")

        def run(idx_hbm, out_hbm, base):
            pltpu.sync_copy(idx_hbm.at[pl.ds(base, E_PER_GW)], idx_v)

            def fire(g, buf, sem):
                off = g * 640
                for b in range(5):
                    pltpu.async_copy(
                        table_hbm.at[idx_v.at[pl.ds(off + b * 128, 128)]],
                        buf.at[pl.ds(b * 128, 128)], sem)

            def drain(buf, sem):
                for b in range(5):
                    pltpu.make_async_copy(
                        table_hbm.at[idx_v.at[pl.ds(b * 128, 128)]],
                        buf.at[pl.ds(b * 128, 128)], sem).wait()

            fire(0, rows0_v, sem0)

            def body(gp, carry):
                g0 = 2 * gp
                fire(g0 + 1, rows1_v, sem1)
                drain(rows0_v, sem0)
                pltpu.sync_copy(rows0_v, out_hbm.at[pl.ds(base + g0 * 640, 640)])

                @pl.when(gp < G_MAIN // 2 - 1)
                def _():
                    fire(g0 + 2, rows0_v, sem0)

                drain(rows1_v, sem1)
                pltpu.sync_copy(rows1_v,
                                out_hbm.at[pl.ds(base + (g0 + 1) * 640, 640)])
                return carry

            lax.fori_loop(0, G_MAIN // 2, body, 0)
            pltpu.async_copy(
                table_hbm.at[idx_v.at[pl.ds(G_MAIN * 640, G_TAIL)]],
                tail_v, sem0).wait()
            pltpu.sync_copy(tail_v,
                            out_hbm.at[pl.ds(base + G_MAIN * 640, G_TAIL)])

        @pl.when(wid < 16)
        def _():
            run(isrc_hbm, osrc_hbm, wid * E_PER_GW)

        @pl.when(wid >= 16)
        def _():
            run(idst_hbm, odst_hbm, (wid - 16) * E_PER_GW)

    return k(table, idx_src, idx_dst)


def _sc_scatter_add(e, idx3, zeros):
    mesh = plsc.VectorSubcoreMesh(core_axis_name="c", subcore_axis_name="s")

    @functools.partial(
        pl.kernel,
        out_type=jax.ShapeDtypeStruct((2, Z_PAD, 16), F32),
        mesh=mesh,
        scratch_types=[
            pltpu.VMEM_SHARED((Z_PAD, 16), F32),
            pltpu.VMEM((IDX_ROWS, 128), I32),
            pltpu.VMEM((512, 16), F32),
            pltpu.VMEM((512, 16), F32),
            pltpu.SemaphoreType.DMA,
            pltpu.SemaphoreType.DMA,
            pltpu.SemaphoreType.DMA,
        ],
        compiler_params=pltpu.CompilerParams(use_tc_tiling_on_sc=False),
    )
    def k(e_hbm, idx_hbm, z_hbm, out_hbm, shared, idx_v, rows0_v, rows1_v,
          seml0, seml1, sems):
        c = lax.axis_index("c")
        s = lax.axis_index("s")
        wid = s * NC + c

        @pl.when(s == 0)
        def _():
            pltpu.sync_copy(z_hbm, shared)

        pltpu.sync_copy(idx_hbm.at[wid], idx_v)
        plsc.subcore_barrier()
        base = wid * PER_W
        nchunk = IDX_ROWS // 4
        npair = nchunk // 2

        def fire_load(t, buf, sem):
            pltpu.async_copy(e_hbm.at[pl.ds(base + t * 512, 512)], buf, sem)

        def drain_load(t, buf, sem):
            pltpu.make_async_copy(e_hbm.at[pl.ds(base + t * 512, 512)],
                                  buf, sem).wait()

        def scatter(t, buf):
            cps = [
                pltpu.async_copy(buf.at[pl.ds(b * 128, 128)],
                                 shared.at[idx_v.at[t * 4 + b]], sems,
                                 add=True)
                for b in range(4)
            ]
            for cp in cps:
                cp.wait()

        fire_load(0, rows0_v, seml0)

        def body(tp, carry):
            t0 = 2 * tp
            fire_load(t0 + 1, rows1_v, seml1)
            drain_load(t0, rows0_v, seml0)
            scatter(t0, rows0_v)
            fire_load(t0 + 2, rows0_v, seml0)
            drain_load(t0 + 1, rows1_v, seml1)
            scatter(t0 + 1, rows1_v)
            return carry

        lax.fori_loop(0, npair, body, 0)
        drain_load(nchunk - 1, rows0_v, seml0)
        scatter(nchunk - 1, rows0_v)
        plsc.subcore_barrier()
        pltpu.sync_copy(shared.at[pl.ds(s * (Z_PAD // NS), Z_PAD // NS)],
                        out_hbm.at[c, pl.ds(s * (Z_PAD // NS), Z_PAD // NS)])

    return k(e, idx3, zeros)


def _relu(h):
    return jnp.maximum(h, 0.0)


def _row(b):
    return b.reshape(1, -1)


def _enc_body(x_ref, w_ref, b_ref, o_ref):
    o_ref[...] = _relu(
        jnp.dot(x_ref[...], w_ref[...], preferred_element_type=F32) + b_ref[...])


def _enc_x(x, w, b):
    n = x.shape[0]
    grid = (n // NODE_BLK,)
    return pl.pallas_call(
        _enc_body,
        grid=grid,
        in_specs=[
            pl.BlockSpec((NODE_BLK, x.shape[1]), lambda i: (i, 0)),
            pl.BlockSpec(w.shape, lambda i: (0, 0)),
            pl.BlockSpec((1, b.shape[-1]), lambda i: (0, 0)),
        ],
        out_specs=pl.BlockSpec((NODE_BLK, w.shape[1]), lambda i: (i, 0)),
        out_shape=jax.ShapeDtypeStruct((n, w.shape[1]), F32),
    )(x, w, _row(b))


def _edge1_body(gs_ref, gd_ref, f_ref, r0_ref, wpe_ref, bpe_ref, ws_ref,
                wd_ref, wp_ref, b1_ref, w2_ref, b2_ref, w3_ref, b3_ref,
                w4_ref, b4_ref, e_ref):
    f_rep = jnp.dot(f_ref[...], r0_ref[...], preferred_element_type=F32,
                    precision=jax.lax.Precision.HIGHEST)
    pe = _relu(f_rep * wpe_ref[...] + bpe_ref[...])
    h = (jnp.dot(gs_ref[...], ws_ref[...], preferred_element_type=F32)
         + jnp.dot(gd_ref[...], wd_ref[...], preferred_element_type=F32)
         + jnp.dot(pe, wp_ref[...], preferred_element_type=F32)
         + b1_ref[...])
    h = _relu(h)
    h = _relu(jnp.dot(h, w2_ref[...], preferred_element_type=F32) + b2_ref[...])
    h = _relu(jnp.dot(h, w3_ref[...], preferred_element_type=F32) + b3_ref[...])
    e_ref[...] = jnp.dot(h, w4_ref[...], preferred_element_type=F32) + b4_ref[...]


def _edge_mlp1(gs, gd, feat, r0, wpe, bpe, ws, wd, wp, b1eff, w2, b2, w3, b3,
               w4, b4):
    grid = (E_PAD // 8 // EDGE_BLK,)
    full = lambda a: pl.BlockSpec(a.shape, lambda i: (0,) * a.ndim)
    return pl.pallas_call(
        _edge1_body,
        grid=grid,
        in_specs=[
            pl.BlockSpec((EDGE_BLK, 128), lambda i: (i, 0)),
            pl.BlockSpec((EDGE_BLK, 128), lambda i: (i, 0)),
            pl.BlockSpec((EDGE_BLK, 8), lambda i: (i, 0)),
            full(r0), full(wpe), full(bpe), full(ws), full(wd), full(wp),
            full(b1eff), full(w2), full(b2), full(w3), full(b3), full(w4),
            full(b4),
        ],
        out_specs=pl.BlockSpec((EDGE_BLK, 128), lambda i: (i, 0)),
        out_shape=jax.ShapeDtypeStruct((E_PAD // 8, 128), F32),
    )(gs, gd, feat, r0, wpe, bpe, ws, wd, wp, b1eff, w2, b2, w3, b3, w4, b4)


def _node1_body(x_ref, p0_ref, p1_ref, wa_ref, wb_ref, b1_ref, w2_ref, b2_ref,
                w3_ref, b3_ref, o_ref):
    agg = p0_ref[0] + p1_ref[0]
    h = (jnp.dot(x_ref[...], wa_ref[...], preferred_element_type=F32)
         + jnp.dot(agg, wb_ref[...], preferred_element_type=F32)
         + b1_ref[...])
    h = _relu(h)
    h = _relu(jnp.dot(h, w2_ref[...], preferred_element_type=F32) + b2_ref[...])
    h = jnp.dot(h, w3_ref[...], preferred_element_type=F32) + b3_ref[...]
    o_ref[...] = jnp.concatenate([h, x_ref[...]], axis=-1)


def _node_mlp1(x, partials, wa, wb, b1eff, w2, b2, w3, b3):
    n = x.shape[0]
    grid = (n // NODE_BLK,)
    full = lambda a: pl.BlockSpec(a.shape, lambda i: (0,) * a.ndim)
    return pl.pallas_call(
        _node1_body,
        grid=grid,
        in_specs=[
            pl.BlockSpec((NODE_BLK, 16), lambda i: (i, 0)),
            pl.BlockSpec((1, NODE_BLK, 16), lambda i: (0, i, 0)),
            pl.BlockSpec((1, NODE_BLK, 16), lambda i: (1, i, 0)),
            full(wa), full(wb), full(b1eff), full(w2), full(b2), full(w3),
            full(b3),
        ],
        out_specs=pl.BlockSpec((NODE_BLK, 32), lambda i: (i, 0)),
        out_shape=jax.ShapeDtypeStruct((n, 32), F32),
    )(x, partials, partials, wa, wb, b1eff, w2, b2, w3, b3)


def _edge2_body(gs_ref, gd_ref, f_ref, r0_ref, wae_ref, bae_ref, ws_ref,
                wd_ref, wp_ref, b1_ref, w2_ref, b2_ref, w3_ref, b3_ref,
                w4_ref, b4_ref, wd1_ref, bd1_ref, wd2_ref, bd2_ref,
                e_ref, eo_ref):
    f_rep = jnp.dot(f_ref[...], r0_ref[...], preferred_element_type=F32,
                    precision=jax.lax.Precision.HIGHEST)
    ae = _relu(f_rep * wae_ref[...] + bae_ref[...])
    h = (jnp.dot(gs_ref[...], ws_ref[...], preferred_element_type=F32)
         + jnp.dot(gd_ref[...], wd_ref[...], preferred_element_type=F32)
         + jnp.dot(ae, wp_ref[...], preferred_element_type=F32)
         + b1_ref[...])
    h = _relu(h)
    h = _relu(jnp.dot(h, w2_ref[...], preferred_element_type=F32) + b2_ref[...])
    h = _relu(jnp.dot(h, w3_ref[...], preferred_element_type=F32) + b3_ref[...])
    e = jnp.dot(h, w4_ref[...], preferred_element_type=F32) + b4_ref[...]
    e_ref[...] = e
    d = _relu(jnp.dot(e, wd1_ref[...], preferred_element_type=F32) + bd1_ref[...])
    d = jnp.dot(d, wd2_ref[...], preferred_element_type=F32) + bd2_ref[...]
    eo_ref[...] = 1.0 / (1.0 + jnp.exp(-d))


def _edge_mlp2(gs, gd, feat, r0, wae, bae, ws, wd, wp, b1, w2, b2, w3, b3,
               w4, b4, wd1, bd1, wd2, bd2):
    grid = (E_PAD // 8 // EDGE_BLK,)
    full = lambda a: pl.BlockSpec(a.shape, lambda i: (0,) * a.ndim)
    return pl.pallas_call(
        _edge2_body,
        grid=grid,
        in_specs=[
            pl.BlockSpec((EDGE_BLK, 256), lambda i: (i, 0)),
            pl.BlockSpec((EDGE_BLK, 256), lambda i: (i, 0)),
            pl.BlockSpec((EDGE_BLK, 8), lambda i: (i, 0)),
            full(r0), full(wae), full(bae), full(ws), full(wd), full(wp),
            full(b1), full(w2), full(b2), full(w3), full(b3), full(w4),
            full(b4), full(wd1), full(bd1), full(wd2), full(bd2),
        ],
        out_specs=[
            pl.BlockSpec((EDGE_BLK, 128), lambda i: (i, 0)),
            pl.BlockSpec((EDGE_BLK, 8), lambda i: (i, 0)),
        ],
        out_shape=[
            jax.ShapeDtypeStruct((E_PAD // 8, 128), F32),
            jax.ShapeDtypeStruct((E_PAD // 8, 8), F32),
        ],
    )(gs, gd, feat, r0, wae, bae, ws, wd, wp, b1, w2, b2, w3, b3, w4, b4,
      wd1, bd1, wd2, bd2)


def _node2_body(x_ref, p0_ref, p1_ref, wa_ref, wb_ref, b1_ref, w2_ref, b2_ref,
                w3_ref, b3_ref, wn1_ref, bn1_ref, wn2_ref, bn2_ref, o_ref):
    agg = p0_ref[0] + p1_ref[0]
    h = (jnp.dot(x_ref[...], wa_ref[...], preferred_element_type=F32)
         + jnp.dot(agg, wb_ref[...], preferred_element_type=F32)
         + b1_ref[...])
    h = _relu(h)
    h = _relu(jnp.dot(h, w2_ref[...], preferred_element_type=F32) + b2_ref[...])
    h = jnp.dot(h, w3_ref[...], preferred_element_type=F32) + b3_ref[...]
    d = _relu(jnp.dot(h, wn1_ref[...], preferred_element_type=F32) + bn1_ref[...])
    o_ref[...] = jnp.dot(d, wn2_ref[...], preferred_element_type=F32) + bn2_ref[...]


def _node_mlp2(x, partials, wa, wb, b1, w2, b2, w3, b3, wn1, bn1, wn2, bn2):
    n = x.shape[0]
    grid = (n // NODE_BLK,)
    full = lambda a: pl.BlockSpec(a.shape, lambda i: (0,) * a.ndim)
    return pl.pallas_call(
        _node2_body,
        grid=grid,
        in_specs=[
            pl.BlockSpec((NODE_BLK, 32), lambda i: (i, 0)),
            pl.BlockSpec((1, NODE_BLK, 16), lambda i: (0, i, 0)),
            pl.BlockSpec((1, NODE_BLK, 16), lambda i: (1, i, 0)),
            full(wa), full(wb), full(b1), full(w2), full(b2), full(w3),
            full(b3), full(wn1), full(bn1), full(wn2), full(bn2),
        ],
        out_specs=pl.BlockSpec((NODE_BLK, 6), lambda i: (i, 0)),
        out_shape=jax.ShapeDtypeStruct((n, 6), F32),
    )(x, partials, partials, wa, wb, b1, w2, b2, w3, b3, wn1, bn1, wn2, bn2)


def kernel(x, node_indexes_for_prediction_edges, prediction_edges_features,
           prediction_global_features, node_indexes_for_association_edges,
           association_edges_features, params):
    p = params

    w_enc, b_enc = p["node_enc"][0]
    wpe, bpe = p["pred_edge_enc"][0]
    wg, bg = p["glob_enc"][0]
    wae, bae = p["assoc_edge_enc"][0]

    u = _relu(prediction_global_features @ wg + bg)

    (w1, b1), (w2, b2), (w3, b3), (w4, b4) = p["tgl1_edge"]
    ws1, wd1_, wp1, wu1 = w1[0:16], w1[16:32], w1[32:48], w1[48:64]
    b1eff = _row(b1) + u @ wu1

    (nw1, nb1), (nw2, nb2), (nw3, nb3) = p["tgl1_node"]
    nwa1, nwb1, nwu1 = nw1[0:16], nw1[16:32], nw1[32:48]
    nb1eff = _row(nb1) + u @ nwu1

    (v1, c1), (v2, c2), (v3, c3), (v4, c4) = p["tgl2_edge"]
    vs1, vd1, vp1 = v1[0:32], v1[32:64], v1[64:80]

    (mw1, mb1), (mw2, mb2), (mw3, mb3) = p["tgl2_node"]
    mwa1, mwb1 = mw1[0:32], mw1[32:48]

    (dn1, dbn1), (dn2, dbn2) = p["node_dec"]
    (de1, dbe1), (de2, dbe2) = p["edge_dec"]

    pe_src = node_indexes_for_prediction_edges[0]
    pe_dst = node_indexes_for_prediction_edges[1]
    ae_src = node_indexes_for_association_edges[0]
    ae_dst = node_indexes_for_association_edges[1]

    pad = jnp.full((E_PAD - N_EDGES,), DUMP_ROW, dtype=pe_dst.dtype)
    pe_dst3 = jnp.concatenate([pe_dst, pad]).reshape(NW, IDX_ROWS, 128)
    ae_dst3 = jnp.concatenate([ae_dst, pad]).reshape(NW, IDX_ROWS, 128)
    zeros = jnp.zeros((Z_PAD, 16), F32)

    enc_x = _enc_x(x, w_enc, b_enc)

    k8 = lambda w: jnp.kron(jnp.eye(8, dtype=F32), w)
    t8 = lambda b: jnp.tile(b.reshape(1, -1), (1, 8))
    r0 = jnp.kron(jnp.eye(8, dtype=F32), jnp.ones((1, 16), F32))

    gs1, gd1 = _sc_gather2(enc_x, pe_src, pe_dst)
    e1 = _edge_mlp1(gs1.reshape(N_EDGES // 8, 128),
                    gd1.reshape(N_EDGES // 8, 128),
                    prediction_edges_features.reshape(N_EDGES // 8, 8),
                    r0, t8(wpe), t8(bpe), k8(ws1), k8(wd1_), k8(wp1),
                    t8(b1eff), k8(w2), t8(b2), k8(w3), t8(b3), k8(w4),
                    t8(b4))

    agg1 = _sc_scatter_add(e1.reshape(E_PAD, 16), pe_dst3, zeros)

    merged = _node_mlp1(enc_x, agg1, nwa1, nwb1, nb1eff,
                        nw2, _row(nb2), nw3, _row(nb3))

    gs2, gd2 = _sc_gather2(merged, ae_src, ae_dst)
    e2, eo = _edge_mlp2(gs2.reshape(N_EDGES // 8, 256),
                        gd2.reshape(N_EDGES // 8, 256),
                        association_edges_features.reshape(N_EDGES // 8, 8),
                        r0, t8(wae), t8(bae), k8(vs1), k8(vd1), k8(vp1),
                        t8(c1), k8(v2), t8(c2), k8(v3), t8(c3), k8(v4),
                        t8(c4), k8(de1), t8(dbe1), k8(de2), t8(dbe2))
    edges_out = eo.reshape(E_PAD, 1)[:N_EDGES]

    agg2 = _sc_scatter_add(e2.reshape(E_PAD, 16), ae_dst3, zeros)

    nodes_out = _node_mlp2(merged, agg2, mwa1, mwb1, _row(mb1),
                           mw2, _row(mb2), mw3, _row(mb3),
                           dn1, _row(dbn1), dn2, _row(dbn2))

    return (nodes_out, edges_out)


# speedup vs baseline: 10.7171x; 1.1357x over previous
import functools

import jax
import jax.numpy as jnp
from jax import lax
from jax.experimental import pallas as pl
from jax.experimental.pallas import tpu as pltpu
from jax.experimental.pallas import tpu_sc as plsc

F32 = jnp.float32
I32 = jnp.int32

N_NODES = 50000
N_EDGES = 800000

NC = 2
NS = 16
NW = NC * NS

E_PAD = 802816
PER_W = E_PAD // NW
IDX_ROWS = PER_W // 128

E_PER_GW = N_EDGES // 16
G_MAIN = E_PER_GW // 640
G_TAIL = E_PER_GW - G_MAIN * 640

Z_PAD = 50048
DUMP_ROW = N_NODES

EDGE_BLK = 1568
NODE_BLK = 2000


def _sc_gather2(table, idx_src, idx_dst):
    d = table.shape[1]
    mesh = plsc.VectorSubcoreMesh(core_axis_name="c", subcore_axis_name="s")

    @functools.partial(
        pl.kernel,
        out_type=[jax.ShapeDtypeStruct((N_EDGES, d), F32),
                  jax.ShapeDtypeStruct((N_EDGES, d), F32)],
        mesh=mesh,
        scratch_types=[
            pltpu.VMEM((E_PER_GW,), I32),
            pltpu.VMEM((640, d), F32),
            pltpu.VMEM((640, d), F32),
            pltpu.VMEM((G_TAIL, d), F32),
            pltpu.SemaphoreType.DMA,
            pltpu.SemaphoreType.DMA,
        ],
        compiler_params=pltpu.CompilerParams(use_tc_tiling_on_sc=False),
    )
    def k(table_hbm, isrc_hbm, idst_hbm, osrc_hbm, odst_hbm,
          idx_v, rows0_v, rows1_v, tail_v, sem0, sem1):
        wid = lax.axis_index("s") * NC + lax.axis_index("c")

        def run(idx_hbm, out_hbm, base):
            pltpu.sync_copy(idx_hbm.at[pl.ds(base, E_PER_GW)], idx_v)

            def fire(g, buf, sem):
                off = g * 640
                for b in range(5):
                    pltpu.async_copy(
                        table_hbm.at[idx_v.at[pl.ds(off + b * 128, 128)]],
                        buf.at[pl.ds(b * 128, 128)], sem)

            def drain(buf, sem):
                for b in range(5):
                    pltpu.make_async_copy(
                        table_hbm.at[idx_v.at[pl.ds(b * 128, 128)]],
                        buf.at[pl.ds(b * 128, 128)], sem).wait()

            fire(0, rows0_v, sem0)

            def body(gp, carry):
                g0 = 2 * gp
                fire(g0 + 1, rows1_v, sem1)
                drain(rows0_v, sem0)
                pltpu.sync_copy(rows0_v, out_hbm.at[pl.ds(base + g0 * 640, 640)])

                @pl.when(gp < G_MAIN // 2 - 1)
                def _():
                    fire(g0 + 2, rows0_v, sem0)

                drain(rows1_v, sem1)
                pltpu.sync_copy(rows1_v,
                                out_hbm.at[pl.ds(base + (g0 + 1) * 640, 640)])
                return carry

            lax.fori_loop(0, G_MAIN // 2, body, 0)
            pltpu.async_copy(
                table_hbm.at[idx_v.at[pl.ds(G_MAIN * 640, G_TAIL)]],
                tail_v, sem0).wait()
            pltpu.sync_copy(tail_v,
                            out_hbm.at[pl.ds(base + G_MAIN * 640, G_TAIL)])

        @pl.when(wid < 16)
        def _():
            run(isrc_hbm, osrc_hbm, wid * E_PER_GW)

        @pl.when(wid >= 16)
        def _():
            run(idst_hbm, odst_hbm, (wid - 16) * E_PER_GW)

    return k(table, idx_src, idx_dst)


def _sc_scatter_add(e, idx3, zeros):
    mesh = plsc.VectorSubcoreMesh(core_axis_name="c", subcore_axis_name="s")

    @functools.partial(
        pl.kernel,
        out_type=jax.ShapeDtypeStruct((2, Z_PAD, 16), F32),
        mesh=mesh,
        scratch_types=[
            pltpu.VMEM_SHARED((Z_PAD, 16), F32),
            pltpu.VMEM((IDX_ROWS, 128), I32),
            pltpu.VMEM((512, 16), F32),
            pltpu.VMEM((512, 16), F32),
            pltpu.SemaphoreType.DMA,
            pltpu.SemaphoreType.DMA,
            pltpu.SemaphoreType.DMA,
        ],
        compiler_params=pltpu.CompilerParams(use_tc_tiling_on_sc=False),
    )
    def k(e_hbm, idx_hbm, z_hbm, out_hbm, shared, idx_v, rows0_v, rows1_v,
          seml0, seml1, sems):
        c = lax.axis_index("c")
        s = lax.axis_index("s")
        wid = s * NC + c

        @pl.when(s == 0)
        def _():
            pltpu.sync_copy(z_hbm, shared)

        pltpu.sync_copy(idx_hbm.at[wid], idx_v)
        plsc.subcore_barrier()
        base = wid * PER_W
        nchunk = IDX_ROWS // 4
        npair = nchunk // 2

        def fire_load(t, buf, sem):
            pltpu.async_copy(e_hbm.at[pl.ds(base + t * 512, 512)], buf, sem)

        def drain_load(t, buf, sem):
            pltpu.make_async_copy(e_hbm.at[pl.ds(base + t * 512, 512)],
                                  buf, sem).wait()

        def scatter(t, buf):
            cps = [
                pltpu.async_copy(buf.at[pl.ds(b * 128, 128)],
                                 shared.at[idx_v.at[t * 4 + b]], sems,
                                 add=True)
                for b in range(4)
            ]
            for cp in cps:
                cp.wait()

        fire_load(0, rows0_v, seml0)

        def body(tp, carry):
            t0 = 2 * tp
            fire_load(t0 + 1, rows1_v, seml1)
            drain_load(t0, rows0_v, seml0)
            scatter(t0, rows0_v)
            fire_load(t0 + 2, rows0_v, seml0)
            drain_load(t0 + 1, rows1_v, seml1)
            scatter(t0 + 1, rows1_v)
            return carry

        lax.fori_loop(0, npair, body, 0)
        drain_load(nchunk - 1, rows0_v, seml0)
        scatter(nchunk - 1, rows0_v)
        plsc.subcore_barrier()
        pltpu.sync_copy(shared.at[pl.ds(s * (Z_PAD // NS), Z_PAD // NS)],
                        out_hbm.at[c, pl.ds(s * (Z_PAD // NS), Z_PAD // NS)])

    return k(e, idx3, zeros)


def _relu(h):
    return jnp.maximum(h, 0.0)


def _row(b):
    return b.reshape(1, -1)


def _enc_body(x_ref, w_ref, b_ref, o_ref):
    o_ref[...] = _relu(
        jnp.dot(x_ref[...], w_ref[...], preferred_element_type=F32) + b_ref[...])


def _enc_x(x, w, b):
    n = x.shape[0]
    grid = (n // NODE_BLK,)
    return pl.pallas_call(
        _enc_body,
        grid=grid,
        in_specs=[
            pl.BlockSpec((NODE_BLK, x.shape[1]), lambda i: (i, 0)),
            pl.BlockSpec(w.shape, lambda i: (0, 0)),
            pl.BlockSpec((1, b.shape[-1]), lambda i: (0, 0)),
        ],
        out_specs=pl.BlockSpec((NODE_BLK, w.shape[1]), lambda i: (i, 0)),
        out_shape=jax.ShapeDtypeStruct((n, w.shape[1]), F32),
    )(x, w, _row(b))


def _edge1_body(gs_ref, gd_ref, f_ref, r0_ref, wpe_ref, bpe_ref, ws_ref,
                wd_ref, wp_ref, b1_ref, w2_ref, b2_ref, w3_ref, b3_ref,
                w4_ref, b4_ref, e_ref):
    f_rep = jnp.dot(f_ref[...], r0_ref[...], preferred_element_type=F32,
                    precision=jax.lax.Precision.HIGHEST)
    pe = _relu(f_rep * wpe_ref[...] + bpe_ref[...])
    h = (jnp.dot(gs_ref[...], ws_ref[...], preferred_element_type=F32)
         + jnp.dot(gd_ref[...], wd_ref[...], preferred_element_type=F32)
         + jnp.dot(pe, wp_ref[...], preferred_element_type=F32)
         + b1_ref[...])
    h = _relu(h)
    h = _relu(jnp.dot(h, w2_ref[...], preferred_element_type=F32) + b2_ref[...])
    h = _relu(jnp.dot(h, w3_ref[...], preferred_element_type=F32) + b3_ref[...])
    e_ref[...] = jnp.dot(h, w4_ref[...], preferred_element_type=F32) + b4_ref[...]


def _edge_mlp1(gs, gd, feat, r0, wpe, bpe, ws, wd, wp, b1eff, w2, b2, w3, b3,
               w4, b4):
    grid = (E_PAD // 8 // EDGE_BLK,)
    full = lambda a: pl.BlockSpec(a.shape, lambda i: (0,) * a.ndim)
    return pl.pallas_call(
        _edge1_body,
        grid=grid,
        in_specs=[
            pl.BlockSpec((EDGE_BLK, 128), lambda i: (i, 0)),
            pl.BlockSpec((EDGE_BLK, 128), lambda i: (i, 0)),
            pl.BlockSpec((EDGE_BLK, 8), lambda i: (i, 0)),
            full(r0), full(wpe), full(bpe), full(ws), full(wd), full(wp),
            full(b1eff), full(w2), full(b2), full(w3), full(b3), full(w4),
            full(b4),
        ],
        out_specs=pl.BlockSpec((EDGE_BLK, 128), lambda i: (i, 0)),
        out_shape=jax.ShapeDtypeStruct((E_PAD // 8, 128), F32),
    )(gs, gd, feat, r0, wpe, bpe, ws, wd, wp, b1eff, w2, b2, w3, b3, w4, b4)


def _node1_body(x_ref, p0_ref, p1_ref, wa_ref, wb_ref, b1_ref, w2_ref, b2_ref,
                w3_ref, b3_ref, o_ref):
    agg = p0_ref[0] + p1_ref[0]
    h = (jnp.dot(x_ref[...], wa_ref[...], preferred_element_type=F32)
         + jnp.dot(agg, wb_ref[...], preferred_element_type=F32)
         + b1_ref[...])
    h = _relu(h)
    h = _relu(jnp.dot(h, w2_ref[...], preferred_element_type=F32) + b2_ref[...])
    o_ref[...] = jnp.dot(h, w3_ref[...], preferred_element_type=F32) + b3_ref[...]


def _node_mlp1(x, partials, wa, wb, b1eff, w2, b2, w3, b3):
    n = x.shape[0]
    grid = (n // NODE_BLK,)
    full = lambda a: pl.BlockSpec(a.shape, lambda i: (0,) * a.ndim)
    return pl.pallas_call(
        _node1_body,
        grid=grid,
        in_specs=[
            pl.BlockSpec((NODE_BLK, 16), lambda i: (i, 0)),
            pl.BlockSpec((1, NODE_BLK, 16), lambda i: (0, i, 0)),
            pl.BlockSpec((1, NODE_BLK, 16), lambda i: (1, i, 0)),
            full(wa), full(wb), full(b1eff), full(w2), full(b2), full(w3),
            full(b3),
        ],
        out_specs=pl.BlockSpec((NODE_BLK, 16), lambda i: (i, 0)),
        out_shape=jax.ShapeDtypeStruct((n, 16), F32),
    )(x, partials, partials, wa, wb, b1eff, w2, b2, w3, b3)


def _edge2_body(gsl_ref, gsh_ref, gdl_ref, gdh_ref, f_ref, r0_ref, wae_ref,
                bae_ref, wsl_ref, wsh_ref, wdl_ref, wdh_ref, wp_ref, b1_ref,
                w2_ref, b2_ref, w3_ref, b3_ref, w4_ref, b4_ref, wd1_ref,
                bd1_ref, wd2_ref, bd2_ref, e_ref, eo_ref):
    f_rep = jnp.dot(f_ref[...], r0_ref[...], preferred_element_type=F32,
                    precision=jax.lax.Precision.HIGHEST)
    ae = _relu(f_rep * wae_ref[...] + bae_ref[...])
    h = (jnp.dot(gsl_ref[...], wsl_ref[...], preferred_element_type=F32)
         + jnp.dot(gsh_ref[...], wsh_ref[...], preferred_element_type=F32)
         + jnp.dot(gdl_ref[...], wdl_ref[...], preferred_element_type=F32)
         + jnp.dot(gdh_ref[...], wdh_ref[...], preferred_element_type=F32)
         + jnp.dot(ae, wp_ref[...], preferred_element_type=F32)
         + b1_ref[...])
    h = _relu(h)
    h = _relu(jnp.dot(h, w2_ref[...], preferred_element_type=F32) + b2_ref[...])
    h = _relu(jnp.dot(h, w3_ref[...], preferred_element_type=F32) + b3_ref[...])
    e = jnp.dot(h, w4_ref[...], preferred_element_type=F32) + b4_ref[...]
    e_ref[...] = e
    d = _relu(jnp.dot(e, wd1_ref[...], preferred_element_type=F32) + bd1_ref[...])
    d = jnp.dot(d, wd2_ref[...], preferred_element_type=F32) + bd2_ref[...]
    eo_ref[...] = 1.0 / (1.0 + jnp.exp(-d))


def _edge_mlp2(gsl, gsh, gdl, gdh, feat, r0, wae, bae, wsl, wsh, wdl, wdh,
               wp, b1, w2, b2, w3, b3, w4, b4, wd1, bd1, wd2, bd2):
    grid = (E_PAD // 8 // EDGE_BLK,)
    full = lambda a: pl.BlockSpec(a.shape, lambda i: (0,) * a.ndim)
    return pl.pallas_call(
        _edge2_body,
        grid=grid,
        in_specs=[
            pl.BlockSpec((EDGE_BLK, 128), lambda i: (i, 0)),
            pl.BlockSpec((EDGE_BLK, 128), lambda i: (i, 0)),
            pl.BlockSpec((EDGE_BLK, 128), lambda i: (i, 0)),
            pl.BlockSpec((EDGE_BLK, 128), lambda i: (i, 0)),
            pl.BlockSpec((EDGE_BLK, 8), lambda i: (i, 0)),
            full(r0), full(wae), full(bae), full(wsl), full(wsh), full(wdl),
            full(wdh), full(wp), full(b1), full(w2), full(b2), full(w3),
            full(b3), full(w4), full(b4), full(wd1), full(bd1), full(wd2),
            full(bd2),
        ],
        out_specs=[
            pl.BlockSpec((EDGE_BLK, 128), lambda i: (i, 0)),
            pl.BlockSpec((EDGE_BLK, 8), lambda i: (i, 0)),
        ],
        out_shape=[
            jax.ShapeDtypeStruct((E_PAD // 8, 128), F32),
            jax.ShapeDtypeStruct((E_PAD // 8, 8), F32),
        ],
    )(gsl, gsh, gdl, gdh, feat, r0, wae, bae, wsl, wsh, wdl, wdh, wp, b1,
      w2, b2, w3, b3, w4, b4, wd1, bd1, wd2, bd2)


def _node2_body(xl_ref, xh_ref, p0_ref, p1_ref, wal_ref, wah_ref, wb_ref,
                b1_ref, w2_ref, b2_ref, w3_ref, b3_ref, wn1_ref, bn1_ref,
                wn2_ref, bn2_ref, o_ref):
    agg = p0_ref[0] + p1_ref[0]
    h = (jnp.dot(xl_ref[...], wal_ref[...], preferred_element_type=F32)
         + jnp.dot(xh_ref[...], wah_ref[...], preferred_element_type=F32)
         + jnp.dot(agg, wb_ref[...], preferred_element_type=F32)
         + b1_ref[...])
    h = _relu(h)
    h = _relu(jnp.dot(h, w2_ref[...], preferred_element_type=F32) + b2_ref[...])
    h = jnp.dot(h, w3_ref[...], preferred_element_type=F32) + b3_ref[...]
    d = _relu(jnp.dot(h, wn1_ref[...], preferred_element_type=F32) + bn1_ref[...])
    o_ref[...] = jnp.dot(d, wn2_ref[...], preferred_element_type=F32) + bn2_ref[...]


def _node_mlp2(xl, xh, partials, wal, wah, wb, b1, w2, b2, w3, b3, wn1, bn1,
               wn2, bn2):
    n = xl.shape[0]
    grid = (n // NODE_BLK,)
    full = lambda a: pl.BlockSpec(a.shape, lambda i: (0,) * a.ndim)
    return pl.pallas_call(
        _node2_body,
        grid=grid,
        in_specs=[
            pl.BlockSpec((NODE_BLK, 16), lambda i: (i, 0)),
            pl.BlockSpec((NODE_BLK, 16), lambda i: (i, 0)),
            pl.BlockSpec((1, NODE_BLK, 16), lambda i: (0, i, 0)),
            pl.BlockSpec((1, NODE_BLK, 16), lambda i: (1, i, 0)),
            full(wal), full(wah), full(wb), full(b1), full(w2), full(b2),
            full(w3), full(b3), full(wn1), full(bn1), full(wn2), full(bn2),
        ],
        out_specs=pl.BlockSpec((NODE_BLK, 6), lambda i: (i, 0)),
        out_shape=jax.ShapeDtypeStruct((n, 6), F32),
    )(xl, xh, partials, partials, wal, wah, wb, b1, w2, b2, w3, b3, wn1,
      bn1, wn2, bn2)


def kernel(x, node_indexes_for_prediction_edges, prediction_edges_features,
           prediction_global_features, node_indexes_for_association_edges,
           association_edges_features, params):
    p = params

    w_enc, b_enc = p["node_enc"][0]
    wpe, bpe = p["pred_edge_enc"][0]
    wg, bg = p["glob_enc"][0]
    wae, bae = p["assoc_edge_enc"][0]

    u = _relu(prediction_global_features @ wg + bg)

    (w1, b1), (w2, b2), (w3, b3), (w4, b4) = p["tgl1_edge"]
    ws1, wd1_, wp1, wu1 = w1[0:16], w1[16:32], w1[32:48], w1[48:64]
    b1eff = _row(b1) + u @ wu1

    (nw1, nb1), (nw2, nb2), (nw3, nb3) = p["tgl1_node"]
    nwa1, nwb1, nwu1 = nw1[0:16], nw1[16:32], nw1[32:48]
    nb1eff = _row(nb1) + u @ nwu1

    (v1, c1), (v2, c2), (v3, c3), (v4, c4) = p["tgl2_edge"]
    vsl, vsh, vdl, vdh, vp1 = (v1[0:16], v1[16:32], v1[32:48], v1[48:64],
                               v1[64:80])

    (mw1, mb1), (mw2, mb2), (mw3, mb3) = p["tgl2_node"]
    mwal, mwah, mwb1 = mw1[0:16], mw1[16:32], mw1[32:48]

    (dn1, dbn1), (dn2, dbn2) = p["node_dec"]
    (de1, dbe1), (de2, dbe2) = p["edge_dec"]

    pe_src = node_indexes_for_prediction_edges[0]
    pe_dst = node_indexes_for_prediction_edges[1]
    ae_src = node_indexes_for_association_edges[0]
    ae_dst = node_indexes_for_association_edges[1]

    pad = jnp.full((E_PAD - N_EDGES,), DUMP_ROW, dtype=pe_dst.dtype)
    pe_dst3 = jnp.concatenate([pe_dst, pad]).reshape(NW, IDX_ROWS, 128)
    ae_dst3 = jnp.concatenate([ae_dst, pad]).reshape(NW, IDX_ROWS, 128)
    zeros = jnp.zeros((Z_PAD, 16), F32)

    enc_x = _enc_x(x, w_enc, b_enc)

    k8 = lambda w: jnp.kron(jnp.eye(8, dtype=F32), w)
    t8 = lambda b: jnp.tile(b.reshape(1, -1), (1, 8))
    r0 = jnp.kron(jnp.eye(8, dtype=F32), jnp.ones((1, 16), F32))

    gs1, gd1 = _sc_gather2(enc_x, pe_src, pe_dst)
    e1 = _edge_mlp1(gs1.reshape(N_EDGES // 8, 128),
                    gd1.reshape(N_EDGES // 8, 128),
                    prediction_edges_features.reshape(N_EDGES // 8, 8),
                    r0, t8(wpe), t8(bpe), k8(ws1), k8(wd1_), k8(wp1),
                    t8(b1eff), k8(w2), t8(b2), k8(w3), t8(b3), k8(w4),
                    t8(b4))

    agg1 = _sc_scatter_add(e1.reshape(E_PAD, 16), pe_dst3, zeros)

    pred = _node_mlp1(enc_x, agg1, nwa1, nwb1, nb1eff,
                      nw2, _row(nb2), nw3, _row(nb3))

    gsl, gdl = _sc_gather2(pred, ae_src, ae_dst)
    gsh, gdh = _sc_gather2(enc_x, ae_src, ae_dst)
    p8 = lambda a: a.reshape(N_EDGES // 8, 128)
    e2, eo = _edge_mlp2(p8(gsl), p8(gsh), p8(gdl), p8(gdh),
                        association_edges_features.reshape(N_EDGES // 8, 8),
                        r0, t8(wae), t8(bae), k8(vsl), k8(vsh), k8(vdl),
                        k8(vdh), k8(vp1), t8(c1), k8(v2), t8(c2), k8(v3),
                        t8(c3), k8(v4), t8(c4), k8(de1), t8(dbe1), k8(de2),
                        t8(dbe2))
    edges_out = eo.reshape(E_PAD, 1)[:N_EDGES]

    agg2 = _sc_scatter_add(e2.reshape(E_PAD, 16), ae_dst3, zeros)

    nodes_out = _node_mlp2(pred, enc_x, agg2, mwal, mwah, mwb1, _row(mb1),
                           mw2, _row(mb2), mw3, _row(mb3),
                           dn1, _row(dbn1), dn2, _row(dbn2))

    return (nodes_out, edges_out)
